# baseline (speedup 1.0000x reference)
import jax
import jax.numpy as jnp
from jax import lax
from jax.experimental import pallas as pl
from jax.experimental.pallas import tpu as pltpu
from jax.experimental.pallas import tpu_sc as plsc

DIM = 64
SEQ = 50
BATCH = 16384
NC, NS = 2, 16
NW = NC * NS
B_PER_W = BATCH // NW
VREGS_B = B_PER_W // 16


def _sc_body(idx_hbm, coef_hbm, out_hbm, idx_v, coef_v, xf_v, stage_v, sem):
    wid = lax.axis_index("s") * NC + lax.axis_index("c")
    b0 = wid * B_PER_W

    pltpu.sync_copy(idx_hbm.at[pl.ds(b0, B_PER_W)], idx_v)
    pltpu.sync_copy(coef_hbm, coef_v)

    iota = lax.iota(jnp.int32, 16)

    def compute_plane(t, buf):
        def stage_x(vb, carry):
            bidx = vb * 16 + iota
            x = plsc.load_gather(idx_v, [bidx, jnp.full((16,), t, jnp.int32)])
            xf_v[pl.ds(vb * 16, 16)] = x.astype(jnp.float32)
            return carry

        lax.fori_loop(0, VREGS_B, stage_x, 0)

        def col(c, carry):
            a = coef_v[c, pl.ds(0, 16)]
            b = coef_v[c, pl.ds(16, 16)]
            q = coef_v[c, pl.ds(32, 16)]
            for vb in range(VREGS_B):
                x = xf_v[pl.ds(vb * 16, 16)]
                stage_v[buf, c, pl.ds(vb * 16, 16)] = a + x * (b + x * q)
            return carry

        lax.fori_loop(0, DIM, col, 0)

    def run():
        def body(t, carry):
            buf = lax.rem(t, 2)

            @pl.when(t >= 2)
            def _():
                pltpu.make_async_copy(
                    stage_v.at[buf],
                    out_hbm.at[t - 2, :, pl.ds(b0, B_PER_W)], sem).wait()

            compute_plane(t, buf)
            pltpu.async_copy(stage_v.at[buf],
                             out_hbm.at[t, :, pl.ds(b0, B_PER_W)], sem)
            return carry

        lax.fori_loop(0, SEQ, body, 0)
        for t in (SEQ - 2, SEQ - 1):
            pltpu.make_async_copy(
                stage_v.at[t % 2], out_hbm.at[t, :, pl.ds(b0, B_PER_W)],
                sem).wait()

    run()


@jax.jit
def _embed(discard_data, table):
    a = table[0]
    q = (table[2] - 2.0 * table[1] + table[0]) * 0.5
    b = table[1] - table[0] - q
    coef = jnp.stack([a, b, q], axis=1)
    coef = jnp.repeat(coef, 16, axis=1)
    mesh = plsc.VectorSubcoreMesh(core_axis_name="c", subcore_axis_name="s")
    out = pl.kernel(
        _sc_body,
        out_type=jax.ShapeDtypeStruct((SEQ, DIM, BATCH), jnp.float32),
        mesh=mesh,
        compiler_params=pltpu.CompilerParams(needs_layout_passes=False,
                                             use_tc_tiling_on_sc=False),
        scratch_types=[
            pltpu.VMEM((B_PER_W, SEQ), jnp.int32),
            pltpu.VMEM((DIM, 48), jnp.float32),
            pltpu.VMEM((B_PER_W,), jnp.float32),
            pltpu.VMEM((2, DIM, B_PER_W), jnp.float32),
            pltpu.SemaphoreType.DMA,
        ],
    )(discard_data, coef)
    return jnp.transpose(out, (2, 0, 1))


def kernel(discard_data, table):
    return _embed(discard_data, table)


# speedup vs baseline: 2.7870x; 2.7870x over previous
import jax
import jax.numpy as jnp
from jax import lax
from jax.experimental import pallas as pl
from jax.experimental.pallas import tpu as pltpu
from jax.experimental.pallas import tpu_sc as plsc

DIM = 64
SEQ = 50
BATCH = 16384
NC, NS = 2, 16
NW = NC * NS
B_PER_W = BATCH // NW
VREGS_B = B_PER_W // 16


def _sc_body(idx_hbm, coef_hbm, out_hbm, idx_v, coef_v, xf_v, stage_v, sem):
    wid = lax.axis_index("s") * NC + lax.axis_index("c")
    b0 = wid * B_PER_W

    pltpu.sync_copy(idx_hbm.at[pl.ds(b0, B_PER_W)], idx_v)
    pltpu.sync_copy(coef_hbm, coef_v)

    iota = lax.iota(jnp.int32, 16)

    def compute_plane(t, buf):
        @plsc.parallel_loop(0, VREGS_B, unroll=4)
        def stage_x(vb):
            bidx = vb * 16 + iota
            x = plsc.load_gather(idx_v, [bidx, jnp.full((16,), t, jnp.int32)])
            xf_v[pl.ds(vb * 16, 16)] = x.astype(jnp.float32)

        @plsc.parallel_loop(0, DIM, unroll=2)
        def col(c):
            a = coef_v[c, pl.ds(0, 16)]
            b = coef_v[c, pl.ds(16, 16)]
            q = coef_v[c, pl.ds(32, 16)]
            for vb in range(VREGS_B):
                x = xf_v[pl.ds(vb * 16, 16)]
                stage_v[buf, c, pl.ds(vb * 16, 16)] = a + x * (b + x * q)

    def run():
        def body(i, carry):
            for par in range(2):
                t = 2 * i + par

                @pl.when(t >= 2)
                def _():
                    pltpu.make_async_copy(
                        stage_v.at[par],
                        out_hbm.at[t - 2, :, pl.ds(b0, B_PER_W)], sem).wait()

                compute_plane(t, par)
                pltpu.async_copy(stage_v.at[par],
                                 out_hbm.at[t, :, pl.ds(b0, B_PER_W)], sem)
            return carry

        lax.fori_loop(0, SEQ // 2, body, 0)
        for t in (SEQ - 2, SEQ - 1):
            pltpu.make_async_copy(
                stage_v.at[t % 2], out_hbm.at[t, :, pl.ds(b0, B_PER_W)],
                sem).wait()

    run()


@jax.jit
def _embed(discard_data, table):
    a = table[0]
    q = (table[2] - 2.0 * table[1] + table[0]) * 0.5
    b = table[1] - table[0] - q
    coef = jnp.stack([a, b, q], axis=1)
    coef = jnp.repeat(coef, 16, axis=1)
    mesh = plsc.VectorSubcoreMesh(core_axis_name="c", subcore_axis_name="s")
    out = pl.kernel(
        _sc_body,
        out_type=jax.ShapeDtypeStruct((SEQ, DIM, BATCH), jnp.float32),
        mesh=mesh,
        compiler_params=pltpu.CompilerParams(needs_layout_passes=False,
                                             use_tc_tiling_on_sc=False),
        scratch_types=[
            pltpu.VMEM((B_PER_W, SEQ), jnp.int32),
            pltpu.VMEM((DIM, 48), jnp.float32),
            pltpu.VMEM((B_PER_W,), jnp.float32),
            pltpu.VMEM((2, DIM, B_PER_W), jnp.float32),
            pltpu.SemaphoreType.DMA,
        ],
    )(discard_data, coef)
    return jnp.transpose(out, (2, 0, 1))


def kernel(discard_data, table):
    return _embed(discard_data, table)


# speedup vs baseline: 6.1695x; 2.2137x over previous
import jax
import jax.numpy as jnp
from jax import lax
from jax.experimental import pallas as pl
from jax.experimental.pallas import tpu as pltpu
from jax.experimental.pallas import tpu_sc as plsc

DIM = 64
SEQ = 50
BATCH = 16384
NC, NS = 2, 16
NW = NC * NS
B_PER_W = BATCH // NW
VREGS_B = B_PER_W // 16


def _sc_body(idx_hbm, coef_hbm, out_hbm, idx_v, coef_v, xf_v, stage_v, sem):
    wid = lax.axis_index("s") * NC + lax.axis_index("c")
    b0 = wid * B_PER_W

    pltpu.sync_copy(idx_hbm.at[pl.ds(b0, B_PER_W)], idx_v)
    pltpu.sync_copy(coef_hbm, coef_v)

    iota = lax.iota(jnp.int32, 16)

    def compute_plane(t, buf):
        @plsc.parallel_loop(0, VREGS_B, unroll=4)
        def stage_x(vb):
            bidx = vb * 16 + iota
            x = plsc.load_gather(idx_v, [bidx, jnp.full((16,), t, jnp.int32)])
            xf_v[pl.ds(vb * 16, 16)] = x.astype(jnp.float32)

        @plsc.parallel_loop(0, DIM, unroll=2)
        def col(c):
            a = coef_v[c, pl.ds(0, 16)]
            b = coef_v[c, pl.ds(16, 16)]
            q = coef_v[c, pl.ds(32, 16)]
            ch = c // 8
            cl = c % 8
            for vb in range(VREGS_B):
                x = xf_v[pl.ds(vb * 16, 16)]
                stage_v[buf, ch, vb // 8, cl,
                        pl.ds((vb % 8) * 16, 16)] = a + x * (b + x * q)

    bh0 = wid * (B_PER_W // 128)

    def run():
        def body(i, carry):
            for par in range(2):
                t = 2 * i + par

                @pl.when(t >= 2)
                def _():
                    pltpu.make_async_copy(
                        stage_v.at[par],
                        out_hbm.at[t - 2, :, pl.ds(bh0, B_PER_W // 128)],
                        sem).wait()

                compute_plane(t, par)
                pltpu.async_copy(
                    stage_v.at[par],
                    out_hbm.at[t, :, pl.ds(bh0, B_PER_W // 128)], sem)
            return carry

        lax.fori_loop(0, SEQ // 2, body, 0)
        for t in (SEQ - 2, SEQ - 1):
            pltpu.make_async_copy(
                stage_v.at[t % 2],
                out_hbm.at[t, :, pl.ds(bh0, B_PER_W // 128)], sem).wait()

    run()


@jax.jit
def _embed(discard_data, table):
    a = table[0]
    q = (table[2] - 2.0 * table[1] + table[0]) * 0.5
    b = table[1] - table[0] - q
    coef = jnp.stack([a, b, q], axis=1)
    coef = jnp.repeat(coef, 16, axis=1)
    mesh = plsc.VectorSubcoreMesh(core_axis_name="c", subcore_axis_name="s")
    out = pl.kernel(
        _sc_body,
        out_type=jax.ShapeDtypeStruct((SEQ, DIM // 8, BATCH // 128, 8, 128),
                                      jnp.float32),
        mesh=mesh,
        compiler_params=pltpu.CompilerParams(needs_layout_passes=False,
                                             use_tc_tiling_on_sc=False),
        scratch_types=[
            pltpu.VMEM((B_PER_W, SEQ), jnp.int32),
            pltpu.VMEM((DIM, 48), jnp.float32),
            pltpu.VMEM((B_PER_W,), jnp.float32),
            pltpu.VMEM((2, DIM // 8, B_PER_W // 128, 8, 128), jnp.float32),
            pltpu.SemaphoreType.DMA,
        ],
    )(discard_data, coef)
    out = jnp.transpose(out, (2, 4, 0, 1, 3))
    return out.reshape(BATCH, SEQ, DIM)


def kernel(discard_data, table):
    return _embed(discard_data, table)
